# baseline (speedup 1.0000x reference)
import functools
import math

import jax
import jax.numpy as jnp
from jax import lax
from jax.experimental import pallas as pl
from jax.experimental.pallas import tpu as pltpu
from jax.experimental.pallas import tpu_sc as plsc

NUM_BUCKETS = 32
MAX_DISTANCE = 128
N_HEADS = 16
Q_LEN = 2048
K_LEN = 2048
DIAGS = Q_LEN + K_LEN - 1

_THRESHOLDS = (12, 16, 23, 32, 46, 64, 91)

_VFULL_PAD = 4112
_SHIFT_LEN = 4096
_LANES = 16
_CHUNK = 8
_ROWS_PER_TILE = Q_LEN // 2


def _vgather(vec, idx):
    return lax.gather(
        vec, idx[:, None],
        dimension_numbers=lax.GatherDimensionNumbers(
            offset_dims=(), collapsed_slice_dims=(0,), start_index_map=(0,)),
        slice_sizes=(1,),
        mode=lax.GatherScatterMode.PROMISE_IN_BOUNDS)


def _bucket_vec(d):
    rp = d - (Q_LEN - 1)
    na = jnp.abs(rp)
    sign_off = jnp.where(rp > 0, 16, 0)
    large = jnp.full((_LANES,), 8, dtype=jnp.int32)
    for t in _THRESHOLDS:
        large = large + jnp.where(na >= t, 1, 0).astype(jnp.int32)
    return sign_off + jnp.where(na < 8, na, large)


def _tile_body(table_hbm, out_hbm, table_v, vfull_v, v8_v, sem):
    head = lax.axis_index("s")
    q0 = lax.axis_index("c") * _ROWS_PER_TILE

    pltpu.sync_copy(table_hbm, table_v)

    lane = lax.iota(jnp.int32, _LANES)
    col_off = pl.multiple_of(head * NUM_BUCKETS, 8)
    col_lo = table_v[pl.ds(col_off, _LANES)]
    col_hi = table_v[pl.ds(col_off + _LANES, _LANES)]

    def diag_step(j, _):
        d = j * _LANES + lane
        b = _bucket_vec(d)
        bl = jnp.bitwise_and(b, 15)
        vals = jnp.where(
            b < _LANES, _vgather(col_lo, bl), _vgather(col_hi, bl))
        vfull_v[pl.ds(j * _LANES, _LANES)] = vals
        return _

    lax.fori_loop(0, _VFULL_PAD // _LANES, diag_step, None, unroll=4)

    out_base = head * (Q_LEN * K_LEN)

    def drain_chunk():
        pltpu.make_async_copy(
            out_hbm.at[pl.ds(0, _CHUNK * K_LEN)],
            v8_v.at[pl.ds(0, _CHUNK * K_LEN)], sem).wait()

    n_chunks = _ROWS_PER_TILE // 8 // _CHUNK

    for ci in range(8):
        s = 7 - ci

        def shift_step(j, _, s=s):
            v8_v[pl.ds(s * _SHIFT_LEN + j * _LANES, _LANES)] = (
                vfull_v[pl.ds(j * _LANES + s, _LANES)])
            return _

        lax.fori_loop(0, _SHIFT_LEN // _LANES, shift_step, None, unroll=8)

        def fire_step(g, _, ci=ci, s=s):
            for r in range(_CHUNK):
                q = q0 + ci + 8 * (g * _CHUNK + r)
                t = (Q_LEN - 1) - q
                a = t - s
                src_off = pl.multiple_of(s * _SHIFT_LEN + a, 8)
                qt = lax.shift_right_logical(q, 3)
                qr = jnp.bitwise_and(q, 7)
                dst_off = pl.multiple_of(
                    out_base + qt * (8 * K_LEN) + qr * 128, 8)
                for kt in range(K_LEN // 128):
                    pltpu.async_copy(
                        v8_v.at[pl.ds(src_off + kt * 128, 128)],
                        out_hbm.at[pl.ds(dst_off + kt * (8 * 128), 128)],
                        sem)
            if ci > 0:
                drain_chunk()
            return _

        lax.fori_loop(0, n_chunks, fire_step, None)

    def tail_step(g, _):
        drain_chunk()
        return _

    lax.fori_loop(0, n_chunks, tail_step, None)


@functools.partial(jax.jit, static_argnums=())
def _rpb(bias_table):
    mesh = plsc.VectorSubcoreMesh(core_axis_name="c", subcore_axis_name="s")
    run = functools.partial(
        pl.kernel,
        mesh=mesh,
        out_type=jax.ShapeDtypeStruct((N_HEADS * Q_LEN * K_LEN,), jnp.float32),
        scratch_types=[
            pltpu.VMEM((NUM_BUCKETS * N_HEADS,), jnp.float32),
            pltpu.VMEM((_VFULL_PAD,), jnp.float32),
            pltpu.VMEM((8 * _SHIFT_LEN,), jnp.float32),
            pltpu.SemaphoreType.DMA,
        ],
    )(_tile_body)
    return run(bias_table.T.reshape(-1))


def kernel(query_length, key_length, bias_table):
    del query_length, key_length
    out = _rpb(bias_table)
    out5 = out.reshape(N_HEADS, Q_LEN // 8, K_LEN // 128, 8, 128)
    return out5.transpose(0, 1, 3, 2, 4).reshape(1, N_HEADS, Q_LEN, K_LEN)


# speedup vs baseline: 1.0346x; 1.0346x over previous
import functools
import math

import jax
import jax.numpy as jnp
from jax import lax
from jax.experimental import pallas as pl
from jax.experimental.pallas import tpu as pltpu
from jax.experimental.pallas import tpu_sc as plsc

NUM_BUCKETS = 32
MAX_DISTANCE = 128
N_HEADS = 16
Q_LEN = 2048
K_LEN = 2048
DIAGS = Q_LEN + K_LEN - 1

_THRESHOLDS = (12, 16, 23, 32, 46, 64, 91)

_VFULL_PAD = 4112
_SHIFT_LEN = 4096
_LANES = 16
_CHUNK = 8
_ROWS_PER_TILE = Q_LEN // 2


def _vgather(vec, idx):
    return lax.gather(
        vec, idx[:, None],
        dimension_numbers=lax.GatherDimensionNumbers(
            offset_dims=(), collapsed_slice_dims=(0,), start_index_map=(0,)),
        slice_sizes=(1,),
        mode=lax.GatherScatterMode.PROMISE_IN_BOUNDS)


def _bucket_vec(d):
    rp = d - (Q_LEN - 1)
    na = jnp.abs(rp)
    sign_off = jnp.where(rp > 0, 16, 0)
    large = jnp.full((_LANES,), 8, dtype=jnp.int32)
    for t in _THRESHOLDS:
        large = large + jnp.where(na >= t, 1, 0).astype(jnp.int32)
    return sign_off + jnp.where(na < 8, na, large)


def _tile_body(table_hbm, out_hbm, table_v, vfull_v, v8_v, sem):
    head = lax.axis_index("s")
    q0 = lax.axis_index("c") * _ROWS_PER_TILE

    pltpu.sync_copy(table_hbm, table_v)

    lane = lax.iota(jnp.int32, _LANES)
    col_off = pl.multiple_of(head * NUM_BUCKETS, 8)
    col_lo = table_v[pl.ds(col_off, _LANES)]
    col_hi = table_v[pl.ds(col_off + _LANES, _LANES)]

    def diag_step(j, _):
        d = j * _LANES + lane
        b = _bucket_vec(d)
        bl = jnp.bitwise_and(b, 15)
        vals = jnp.where(
            b < _LANES, _vgather(col_lo, bl), _vgather(col_hi, bl))
        vfull_v[pl.ds(j * _LANES, _LANES)] = vals
        return _

    lax.fori_loop(0, _VFULL_PAD // _LANES, diag_step, None, unroll=4)

    for s in range(8):
        def shift_step(j, _, s=s):
            v8_v[pl.ds(s * _SHIFT_LEN + j * _LANES, _LANES)] = (
                vfull_v[pl.ds(j * _LANES + s, _LANES)])
            return _

        lax.fori_loop(0, _SHIFT_LEN // _LANES, shift_step, None, unroll=8)

    out_base = head * (Q_LEN * K_LEN)

    def drain_chunk():
        pltpu.make_async_copy(
            out_hbm.at[pl.ds(0, _CHUNK * K_LEN)],
            v8_v.at[pl.ds(0, _CHUNK * K_LEN)], sem).wait()

    def dma_step(g, _):
        for r in range(_CHUNK):
            q = q0 + g * _CHUNK + r
            t = (Q_LEN - 1) - q
            s = jnp.bitwise_and(t, 7)
            a = t - s
            src_off = pl.multiple_of(s * _SHIFT_LEN + a, 8)
            qt = lax.shift_right_logical(q, 3)
            qr = jnp.bitwise_and(q, 7)
            dst_off = pl.multiple_of(
                out_base + qt * (8 * K_LEN) + qr * 128, 8)
            for kt in range(K_LEN // 128):
                pltpu.async_copy(
                    v8_v.at[pl.ds(src_off + kt * 128, 128)],
                    out_hbm.at[pl.ds(dst_off + kt * (8 * 128), 128)],
                    sem)

        @pl.when(g > 1)
        def _drain_prev():
            drain_chunk()

        return _

    lax.fori_loop(0, _ROWS_PER_TILE // _CHUNK, dma_step, None)
    drain_chunk()
    drain_chunk()


@functools.partial(jax.jit, static_argnums=())
def _rpb(bias_table):
    mesh = plsc.VectorSubcoreMesh(core_axis_name="c", subcore_axis_name="s")
    run = functools.partial(
        pl.kernel,
        mesh=mesh,
        out_type=jax.ShapeDtypeStruct((N_HEADS * Q_LEN * K_LEN,), jnp.float32),
        scratch_types=[
            pltpu.VMEM((NUM_BUCKETS * N_HEADS,), jnp.float32),
            pltpu.VMEM((_VFULL_PAD,), jnp.float32),
            pltpu.VMEM((8 * _SHIFT_LEN,), jnp.float32),
            pltpu.SemaphoreType.DMA,
        ],
    )(_tile_body)
    return run(bias_table.T.reshape(-1))


def kernel(query_length, key_length, bias_table):
    del query_length, key_length
    out = _rpb(bias_table)
    out5 = out.reshape(N_HEADS, Q_LEN // 8, K_LEN // 128, 8, 128)
    return out5.transpose(0, 1, 3, 2, 4).reshape(1, N_HEADS, Q_LEN, K_LEN)
